# baseline (speedup 1.0000x reference)
import functools

import jax
import jax.numpy as jnp
from jax import lax
from jax.experimental import pallas as pl
from jax.experimental.pallas import tpu as pltpu
from jax.experimental.pallas import tpu_sc as plsc

N_ING = 100000
N_TASTE = 10000
E = 625000
D = 128

NC = 2
NS = 16
NW = NC * NS
L = 16

CHUNK = 2048
N_CHUNKS = (E + CHUNK - 1) // CHUNK
E_PAD = N_CHUNKS * CHUNK
ROWS_CH = E_PAD // 128
CH_ROWS = CHUNK // 128
T_PAD = 10240
T_SL = T_PAD // NS
T_ACC = 10112
T_SL2 = T_ACC // NS
BATCH = 128
NB = CHUNK // BATCH

_mesh = plsc.VectorSubcoreMesh(core_axis_name="c", subcore_axis_name="s")
_sc_params = pltpu.CompilerParams(needs_layout_passes=False)


def _score_body(x_ref, w_ref, b_ref, att_ref, a_ref, amax_ref):
    i = pl.program_id(0)
    vr = jax.lax.dot_general(att_ref[...], w_ref[...], (((1,), (1,)), ((), ())),
                             preferred_element_type=jnp.float32)
    c0 = jnp.sum(b_ref[...] * att_ref[...])
    a = jnp.sum(x_ref[...] * vr, axis=1)
    a = a + c0
    a_ref[...] = a.reshape(a_ref.shape)
    m = jnp.max(a)

    @pl.when(i == 0)
    def _():
        amax_ref[0, 0] = m

    @pl.when(i > 0)
    def _():
        amax_ref[0, 0] = jnp.maximum(amax_ref[0, 0], m)


def _score(x, w, b2, att2, n_rows, blk):
    grid = (n_rows // blk,)
    return pl.pallas_call(
        _score_body,
        grid=grid,
        in_specs=[
            pl.BlockSpec((blk, D), lambda i: (i, 0)),
            pl.BlockSpec((D, D), lambda i: (0, 0)),
            pl.BlockSpec((1, D), lambda i: (0, 0)),
            pl.BlockSpec((1, D), lambda i: (0, 0)),
        ],
        out_specs=[
            pl.BlockSpec((1, 1, blk), lambda i: (i, 0, 0)),
            pl.BlockSpec((1, 1), lambda i: (0, 0),
                         memory_space=pltpu.SMEM),
        ],
        out_shape=[
            jax.ShapeDtypeStruct((n_rows // blk, 1, blk), jnp.float32),
            jax.ShapeDtypeStruct((1, 1), jnp.float32),
        ],
    )(x, w, b2, att2)


def _h_body(x_ref, w_ref, b_ref, h_ref):
    h_ref[...] = jnp.dot(x_ref[...], w_ref[...],
                         preferred_element_type=jnp.float32) + b_ref[...]


def _h_proj(x, w, b2, n_rows, blk):
    return pl.pallas_call(
        _h_body,
        grid=(n_rows // blk,),
        in_specs=[
            pl.BlockSpec((blk, D), lambda i: (i, 0)),
            pl.BlockSpec((D, D), lambda i: (0, 0)),
            pl.BlockSpec((1, D), lambda i: (0, 0)),
        ],
        out_specs=pl.BlockSpec((blk, D), lambda i: (i, 0)),
        out_shape=jax.ShapeDtypeStruct((n_rows, D), jnp.float32),
    )(x, w, b2)


def _sck1_body(src_hbm, dst_hbm, asrc_hbm, adst_hbm, m_hbm,
               ex_hbm, denom_hbm,
               asrc_v, adst_v, m_v, srcb, dstb, exb, zbuf, denom_sh,
               se, sdn):
    c = lax.axis_index("c")
    s = lax.axis_index("s")
    wid = s * NC + c

    pltpu.sync_copy(asrc_hbm, asrc_v)
    pltpu.sync_copy(adst_hbm, adst_v)
    pltpu.sync_copy(m_hbm, m_v)
    m = m_v[...][0]

    z16 = jnp.zeros((L,), jnp.float32)
    for k in range(T_SL // L):
        zbuf[pl.ds(k * L, L)] = z16
    pltpu.sync_copy(zbuf, denom_sh.at[pl.ds(s * T_SL, T_SL)])
    plsc.subcore_barrier()

    nchunks = (N_CHUNKS - wid + NW - 1) // NW

    def chunk_body(ci, carry):
        ch = wid + ci * NW
        base_row = ch * CH_ROWS
        pltpu.sync_copy(src_hbm.at[pl.ds(base_row, CH_ROWS)], srcb)
        pltpu.sync_copy(dst_hbm.at[pl.ds(base_row, CH_ROWS)], dstb)

        def row_body(row, rcarry):
            for col in range(8):
                si = srcb[row, pl.ds(col * L, L)]
                di = dstb[row, pl.ds(col * L, L)]
                av = plsc.load_gather(asrc_v, [si])
                dv = plsc.load_gather(adst_v, [di])
                al = av + dv
                al = jnp.where(al >= 0.0, al, 0.2 * al)
                ex = jnp.exp(al - m)
                gidx = (base_row + row) * 128 + col * L + lax.iota(jnp.int32, L)
                ex = jnp.where(gidx < E, ex, 0.0)
                exb[row, pl.ds(col * L, L)] = ex
            return rcarry

        lax.fori_loop(0, CH_ROWS, row_body, 0)
        exd = pltpu.async_copy(exb, ex_hbm.at[pl.ds(base_row, CH_ROWS)], se)
        dds = [pltpu.async_copy(exb.at[row], denom_sh.at[dstb.at[row]],
                                sdn, add=True) for row in range(CH_ROWS)]
        exd.wait()
        for dsc in dds:
            dsc.wait()
        return carry

    lax.fori_loop(0, nchunks, chunk_body, 0)
    plsc.subcore_barrier()
    pltpu.sync_copy(denom_sh.at[pl.ds(s * T_SL, T_SL)],
                    denom_hbm.at[c, pl.ds(s * T_SL, T_SL)])


_sck1 = functools.partial(
    pl.kernel,
    out_type=[
        jax.ShapeDtypeStruct((ROWS_CH, 128), jnp.float32),
        jax.ShapeDtypeStruct((NC, T_PAD), jnp.float32),
    ],
    mesh=_mesh,
    compiler_params=_sc_params,
    scratch_types=[
        pltpu.VMEM((N_ING,), jnp.float32),
        pltpu.VMEM((N_TASTE,), jnp.float32),
        pltpu.VMEM((L,), jnp.float32),
        pltpu.VMEM((CH_ROWS, 128), jnp.int32),
        pltpu.VMEM((CH_ROWS, 128), jnp.int32),
        pltpu.VMEM((CH_ROWS, 128), jnp.float32),
        pltpu.VMEM((T_SL,), jnp.float32),
        pltpu.VMEM_SHARED((T_PAD,), jnp.float32),
        pltpu.SemaphoreType.DMA,
        pltpu.SemaphoreType.DMA,
    ],
)(_sck1_body)


def _scale_batch(rows_ref, exb, b):
    @plsc.parallel_loop(0, BATCH // L, unroll=2)
    def _(g):
        sv = exb[b, pl.ds(g * L, L)]
        for k in range(L):
            svb = jnp.full((L,), sv[k], jnp.float32)
            e = g * L + k
            for j in range(8):
                rows_ref[e, pl.ds(j * L, L)] = rows_ref[e, pl.ds(j * L, L)] * svb


def _sck2_body(src_hbm, dst_hbm, ex_hbm, denom_hbm, denom3_hbm, h_hbm,
               out_hbm,
               r_v, srcb, dstb, exb, rows0, rows1, acc_sh,
               sg0, sg1, ss0, ss1):
    c = lax.axis_index("c")
    s = lax.axis_index("s")
    wid = s * NC + c

    pltpu.sync_copy(denom_hbm.at[0], r_v)
    pltpu.sync_copy(denom3_hbm.at[1], rows1.at[pl.ds(0, T_PAD // 128)])

    @plsc.parallel_loop(0, T_PAD // 128, unroll=2)
    def _(row):
        for col in range(8):
            off = row * 128 + col * L
            dsum = r_v[pl.ds(off, L)] + rows1[row, pl.ds(col * L, L)]
            r_v[pl.ds(off, L)] = 1.0 / jnp.maximum(dsum, 1e-16)

    z16 = jnp.zeros((L,), jnp.float32)

    def zrow_body(e, carry):
        for j in range(8):
            rows0[e, pl.ds(j * L, L)] = z16
        return carry

    lax.fori_loop(0, BATCH, zrow_body, 0)
    for k in range(4):
        pltpu.sync_copy(rows0, acc_sh.at[pl.ds(s * T_SL2 + k * BATCH, BATCH)])
    pltpu.sync_copy(rows0.at[pl.ds(0, T_SL2 - 4 * BATCH)],
                    acc_sh.at[pl.ds(s * T_SL2 + 4 * BATCH, T_SL2 - 4 * BATCH)])
    plsc.subcore_barrier()

    nchunks = (N_CHUNKS - wid + NW - 1) // NW
    rows = (rows0, rows1)
    sgs = (sg0, sg1)
    sss = (ss0, ss1)

    def chunk_body(ci, carry):
        ch = wid + ci * NW
        base_row = ch * CH_ROWS
        pltpu.sync_copy(src_hbm.at[pl.ds(base_row, CH_ROWS)], srcb)
        pltpu.sync_copy(dst_hbm.at[pl.ds(base_row, CH_ROWS)], dstb)
        pltpu.sync_copy(ex_hbm.at[pl.ds(base_row, CH_ROWS)], exb)
        def srow_body(row, rcarry):
            for col in range(8):
                di = dstb[row, pl.ds(col * L, L)]
                rv = plsc.load_gather(r_v, [di])
                exb[row, pl.ds(col * L, L)] = exb[row, pl.ds(col * L, L)] * rv
            return rcarry

        lax.fori_loop(0, CH_ROWS, srow_body, 0)
        pltpu.async_copy(h_hbm.at[srcb.at[0]], rows0, sg0)

        def pair_body(i, pcarry):
            b0 = i * 2
            pltpu.make_async_copy(h_hbm.at[srcb.at[0]], rows0, sg0).wait()

            @pl.when(i > 0)
            def _():
                pltpu.make_async_copy(rows1, acc_sh.at[dstb.at[0]], ss1).wait()

            pltpu.async_copy(h_hbm.at[srcb.at[b0 + 1]], rows1, sg1)
            _scale_batch(rows0, exb, b0)
            pltpu.async_copy(rows0, acc_sh.at[dstb.at[b0]], ss0, add=True)
            pltpu.make_async_copy(h_hbm.at[srcb.at[0]], rows1, sg1).wait()

            @pl.when(i + 1 < NB // 2)
            def _():
                pltpu.make_async_copy(rows0, acc_sh.at[dstb.at[0]], ss0).wait()
                pltpu.async_copy(h_hbm.at[srcb.at[b0 + 2]], rows0, sg0)

            _scale_batch(rows1, exb, b0 + 1)
            pltpu.async_copy(rows1, acc_sh.at[dstb.at[b0 + 1]], ss1, add=True)
            return pcarry

        lax.fori_loop(0, NB // 2, pair_body, 0)
        pltpu.make_async_copy(rows0, acc_sh.at[dstb.at[0]], ss0).wait()
        pltpu.make_async_copy(rows1, acc_sh.at[dstb.at[0]], ss1).wait()
        return carry

    lax.fori_loop(0, nchunks, chunk_body, 0)
    plsc.subcore_barrier()
    for k in range(4):
        pltpu.sync_copy(acc_sh.at[pl.ds(s * T_SL2 + k * BATCH, BATCH)],
                        out_hbm.at[c, pl.ds(s * T_SL2 + k * BATCH, BATCH)])
    pltpu.sync_copy(acc_sh.at[pl.ds(s * T_SL2 + 4 * BATCH, T_SL2 - 4 * BATCH)],
                    out_hbm.at[c, pl.ds(s * T_SL2 + 4 * BATCH,
                                        T_SL2 - 4 * BATCH)])


_sck2 = functools.partial(
    pl.kernel,
    out_type=jax.ShapeDtypeStruct((NC, T_ACC, D), jnp.float32),
    mesh=_mesh,
    compiler_params=_sc_params,
    scratch_types=[
        pltpu.VMEM((T_PAD,), jnp.float32),
        pltpu.VMEM((CH_ROWS, 128), jnp.int32),
        pltpu.VMEM((CH_ROWS, 128), jnp.int32),
        pltpu.VMEM((CH_ROWS, 128), jnp.float32),
        pltpu.VMEM((BATCH, D), jnp.float32),
        pltpu.VMEM((BATCH, D), jnp.float32),
        pltpu.VMEM_SHARED((T_ACC, D), jnp.float32),
        pltpu.SemaphoreType.DMA,
        pltpu.SemaphoreType.DMA,
        pltpu.SemaphoreType.DMA,
        pltpu.SemaphoreType.DMA,
    ],
)(_sck2_body)


def _bn_body(o2_ref, xt_ref, g_ref, bt_ref, out_ref):
    agg = o2_ref[0, :N_TASTE, :] + o2_ref[1, :N_TASTE, :]
    y = jnp.maximum(agg, 0.0) + xt_ref[...]
    mean = jnp.mean(y, axis=0, keepdims=True)
    var = jnp.mean(jnp.square(y - mean), axis=0, keepdims=True)
    yn = (y - mean) / jnp.sqrt(var + 1e-5) * g_ref[...] + bt_ref[...]
    out_ref[...] = jnp.maximum(yn, 0.0)


def _bn(out2, x_taste, g2, bt2):
    return pl.pallas_call(
        _bn_body,
        out_shape=jax.ShapeDtypeStruct((N_TASTE, D), jnp.float32),
    )(out2, x_taste, g2, bt2)


def kernel(x_ingredient, x_taste, edge_src, edge_dst, W_ing, b_ing,
           W_taste, b_taste, att_src, att_dst, Wk, bk, q, gamma, beta):
    del Wk, bk, q

    pad = jnp.zeros((E_PAD - E,), jnp.int32)
    src2 = jnp.concatenate([edge_src, pad]).reshape(ROWS_CH, 128)
    dst2 = jnp.concatenate([edge_dst, pad]).reshape(ROWS_CH, 128)

    a_src, amax_s = _score(
        x_ingredient, W_ing, b_ing.reshape(1, D), att_src.reshape(1, D),
        N_ING, 4000)
    a_src = a_src.reshape(N_ING)
    a_dst, amax_d = _score(
        x_taste, W_taste, b_taste.reshape(1, D), att_dst.reshape(1, D),
        N_TASTE, 2000)
    a_dst = a_dst.reshape(N_TASTE)
    h_ing = _h_proj(x_ingredient, W_ing, b_ing.reshape(1, D), N_ING, 4000)

    mb = amax_s[0, 0] + amax_d[0, 0]
    mb = jnp.where(mb >= 0.0, mb, 0.2 * mb)
    m_vec = jnp.full((L,), mb, jnp.float32)

    ex2, denom2 = _sck1(src2, dst2, a_src, a_dst, m_vec)
    denom3 = denom2.reshape(NC, T_PAD // 128, 128)
    out2 = _sck2(src2, dst2, ex2, denom2, denom3, h_ing)
    out_t = _bn(out2, x_taste, gamma.reshape(1, D), beta.reshape(1, D))
    return (x_ingredient, out_t)


# speedup vs baseline: 1.0192x; 1.0192x over previous
import functools

import jax
import jax.numpy as jnp
from jax import lax
from jax.experimental import pallas as pl
from jax.experimental.pallas import tpu as pltpu
from jax.experimental.pallas import tpu_sc as plsc

N_ING = 100000
N_TASTE = 10000
E = 625000
D = 128

NC = 2
NS = 16
NW = NC * NS
L = 16

CHUNK = 2048
N_CHUNKS = (E + CHUNK - 1) // CHUNK
E_PAD = N_CHUNKS * CHUNK
ROWS_CH = E_PAD // 128
CH_ROWS = CHUNK // 128
T_PAD = 10240
T_SL = T_PAD // NS
T_ACC = 10112
T_SL2 = T_ACC // NS
BATCH = 128
NB = CHUNK // BATCH

_mesh = plsc.VectorSubcoreMesh(core_axis_name="c", subcore_axis_name="s")
_sc_params = pltpu.CompilerParams(needs_layout_passes=False)


def _score_body(x_ref, w_ref, b_ref, att_ref, a_ref, amax_ref):
    i = pl.program_id(0)
    vr = jax.lax.dot_general(att_ref[...], w_ref[...], (((1,), (1,)), ((), ())),
                             preferred_element_type=jnp.float32,
                             precision=jax.lax.Precision.HIGHEST)
    c0 = jnp.sum(b_ref[...] * att_ref[...])
    a = jnp.sum(x_ref[...] * vr, axis=1)
    a = a + c0
    a_ref[...] = a.reshape(a_ref.shape)
    m = jnp.max(a)

    @pl.when(i == 0)
    def _():
        amax_ref[0, 0] = m

    @pl.when(i > 0)
    def _():
        amax_ref[0, 0] = jnp.maximum(amax_ref[0, 0], m)


def _score(x, w, b2, att2, n_rows, blk):
    grid = (n_rows // blk,)
    return pl.pallas_call(
        _score_body,
        grid=grid,
        in_specs=[
            pl.BlockSpec((blk, D), lambda i: (i, 0)),
            pl.BlockSpec((D, D), lambda i: (0, 0)),
            pl.BlockSpec((1, D), lambda i: (0, 0)),
            pl.BlockSpec((1, D), lambda i: (0, 0)),
        ],
        out_specs=[
            pl.BlockSpec((1, 1, blk), lambda i: (i, 0, 0)),
            pl.BlockSpec((1, 1), lambda i: (0, 0),
                         memory_space=pltpu.SMEM),
        ],
        out_shape=[
            jax.ShapeDtypeStruct((n_rows // blk, 1, blk), jnp.float32),
            jax.ShapeDtypeStruct((1, 1), jnp.float32),
        ],
    )(x, w, b2, att2)


def _h_body(x_ref, w_ref, b_ref, h_ref):
    h_ref[...] = jnp.dot(x_ref[...], w_ref[...],
                         preferred_element_type=jnp.float32,
                         precision=jax.lax.Precision.HIGHEST) + b_ref[...]


def _h_proj(x, w, b2, n_rows, blk):
    return pl.pallas_call(
        _h_body,
        grid=(n_rows // blk,),
        in_specs=[
            pl.BlockSpec((blk, D), lambda i: (i, 0)),
            pl.BlockSpec((D, D), lambda i: (0, 0)),
            pl.BlockSpec((1, D), lambda i: (0, 0)),
        ],
        out_specs=pl.BlockSpec((blk, D), lambda i: (i, 0)),
        out_shape=jax.ShapeDtypeStruct((n_rows, D), jnp.float32),
    )(x, w, b2)


def _sck1_body(src_hbm, dst_hbm, asrc_hbm, adst_hbm, m_hbm,
               ex_hbm, denom_hbm,
               asrc_v, adst_v, m_v, srcb, dstb, exb, zbuf, denom_sh,
               se, sdn):
    c = lax.axis_index("c")
    s = lax.axis_index("s")
    wid = s * NC + c

    pltpu.sync_copy(asrc_hbm, asrc_v)
    pltpu.sync_copy(adst_hbm, adst_v)
    pltpu.sync_copy(m_hbm, m_v)
    m = m_v[...][0]

    z16 = jnp.zeros((L,), jnp.float32)
    for k in range(T_SL // L):
        zbuf[pl.ds(k * L, L)] = z16
    pltpu.sync_copy(zbuf, denom_sh.at[pl.ds(s * T_SL, T_SL)])
    plsc.subcore_barrier()

    nchunks = (N_CHUNKS - wid + NW - 1) // NW

    def chunk_body(ci, carry):
        ch = wid + ci * NW
        base_row = ch * CH_ROWS
        pltpu.sync_copy(src_hbm.at[pl.ds(base_row, CH_ROWS)], srcb)
        pltpu.sync_copy(dst_hbm.at[pl.ds(base_row, CH_ROWS)], dstb)

        def row_body(row, rcarry):
            for col in range(8):
                si = srcb[row, pl.ds(col * L, L)]
                di = dstb[row, pl.ds(col * L, L)]
                av = plsc.load_gather(asrc_v, [si])
                dv = plsc.load_gather(adst_v, [di])
                al = av + dv
                al = jnp.where(al >= 0.0, al, 0.2 * al)
                ex = jnp.exp(al - m)
                gidx = (base_row + row) * 128 + col * L + lax.iota(jnp.int32, L)
                ex = jnp.where(gidx < E, ex, 0.0)
                exb[row, pl.ds(col * L, L)] = ex
            return rcarry

        lax.fori_loop(0, CH_ROWS, row_body, 0)
        exd = pltpu.async_copy(exb, ex_hbm.at[pl.ds(base_row, CH_ROWS)], se)
        dds = [pltpu.async_copy(exb.at[row], denom_sh.at[dstb.at[row]],
                                sdn, add=True) for row in range(CH_ROWS)]
        exd.wait()
        for dsc in dds:
            dsc.wait()
        return carry

    lax.fori_loop(0, nchunks, chunk_body, 0)
    plsc.subcore_barrier()
    pltpu.sync_copy(denom_sh.at[pl.ds(s * T_SL, T_SL)],
                    denom_hbm.at[c, pl.ds(s * T_SL, T_SL)])


_sck1 = functools.partial(
    pl.kernel,
    out_type=[
        jax.ShapeDtypeStruct((ROWS_CH, 128), jnp.float32),
        jax.ShapeDtypeStruct((NC, T_PAD), jnp.float32),
    ],
    mesh=_mesh,
    compiler_params=_sc_params,
    scratch_types=[
        pltpu.VMEM((N_ING,), jnp.float32),
        pltpu.VMEM((N_TASTE,), jnp.float32),
        pltpu.VMEM((L,), jnp.float32),
        pltpu.VMEM((CH_ROWS, 128), jnp.int32),
        pltpu.VMEM((CH_ROWS, 128), jnp.int32),
        pltpu.VMEM((CH_ROWS, 128), jnp.float32),
        pltpu.VMEM((T_SL,), jnp.float32),
        pltpu.VMEM_SHARED((T_PAD,), jnp.float32),
        pltpu.SemaphoreType.DMA,
        pltpu.SemaphoreType.DMA,
    ],
)(_sck1_body)


def _scale_batch(rows_ref, exb, b):
    @plsc.parallel_loop(0, BATCH // L, unroll=2)
    def _(g):
        sv = exb[b, pl.ds(g * L, L)]
        for k in range(L):
            svb = jnp.full((L,), sv[k], jnp.float32)
            e = g * L + k
            for j in range(8):
                rows_ref[e, pl.ds(j * L, L)] = rows_ref[e, pl.ds(j * L, L)] * svb


def _sck2_body(src_hbm, dst_hbm, ex_hbm, denom_hbm, denom3_hbm, h_hbm,
               out_hbm,
               r_v, srcb, dstb, exb, rows0, rows1, acc_sh,
               sg0, sg1, ss0, ss1):
    c = lax.axis_index("c")
    s = lax.axis_index("s")
    wid = s * NC + c

    pltpu.sync_copy(denom_hbm.at[0], r_v)
    pltpu.sync_copy(denom3_hbm.at[1], rows1.at[pl.ds(0, T_PAD // 128)])

    @plsc.parallel_loop(0, T_PAD // 128, unroll=2)
    def _(row):
        for col in range(8):
            off = row * 128 + col * L
            dsum = r_v[pl.ds(off, L)] + rows1[row, pl.ds(col * L, L)]
            r_v[pl.ds(off, L)] = 1.0 / jnp.maximum(dsum, 1e-16)

    z16 = jnp.zeros((L,), jnp.float32)

    def zrow_body(e, carry):
        for j in range(8):
            rows0[e, pl.ds(j * L, L)] = z16
        return carry

    lax.fori_loop(0, BATCH, zrow_body, 0)
    for k in range(4):
        pltpu.sync_copy(rows0, acc_sh.at[pl.ds(s * T_SL2 + k * BATCH, BATCH)])
    pltpu.sync_copy(rows0.at[pl.ds(0, T_SL2 - 4 * BATCH)],
                    acc_sh.at[pl.ds(s * T_SL2 + 4 * BATCH, T_SL2 - 4 * BATCH)])
    plsc.subcore_barrier()

    nchunks = (N_CHUNKS - wid + NW - 1) // NW
    rows = (rows0, rows1)
    sgs = (sg0, sg1)
    sss = (ss0, ss1)

    def chunk_body(ci, carry):
        ch = wid + ci * NW
        base_row = ch * CH_ROWS
        pltpu.sync_copy(src_hbm.at[pl.ds(base_row, CH_ROWS)], srcb)
        pltpu.sync_copy(dst_hbm.at[pl.ds(base_row, CH_ROWS)], dstb)
        pltpu.sync_copy(ex_hbm.at[pl.ds(base_row, CH_ROWS)], exb)
        def srow_body(row, rcarry):
            for col in range(8):
                di = dstb[row, pl.ds(col * L, L)]
                rv = plsc.load_gather(r_v, [di])
                exb[row, pl.ds(col * L, L)] = exb[row, pl.ds(col * L, L)] * rv
            return rcarry

        lax.fori_loop(0, CH_ROWS, srow_body, 0)
        pltpu.async_copy(h_hbm.at[srcb.at[0]], rows0, sg0)

        def pair_body(i, pcarry):
            b0 = i * 2
            pltpu.make_async_copy(h_hbm.at[srcb.at[0]], rows0, sg0).wait()

            @pl.when(i > 0)
            def _():
                pltpu.make_async_copy(rows1, acc_sh.at[dstb.at[0]], ss1).wait()

            pltpu.async_copy(h_hbm.at[srcb.at[b0 + 1]], rows1, sg1)
            _scale_batch(rows0, exb, b0)
            pltpu.async_copy(rows0, acc_sh.at[dstb.at[b0]], ss0, add=True)
            pltpu.make_async_copy(h_hbm.at[srcb.at[0]], rows1, sg1).wait()

            @pl.when(i + 1 < NB // 2)
            def _():
                pltpu.make_async_copy(rows0, acc_sh.at[dstb.at[0]], ss0).wait()
                pltpu.async_copy(h_hbm.at[srcb.at[b0 + 2]], rows0, sg0)

            _scale_batch(rows1, exb, b0 + 1)
            pltpu.async_copy(rows1, acc_sh.at[dstb.at[b0 + 1]], ss1, add=True)
            return pcarry

        lax.fori_loop(0, NB // 2, pair_body, 0)
        pltpu.make_async_copy(rows0, acc_sh.at[dstb.at[0]], ss0).wait()
        pltpu.make_async_copy(rows1, acc_sh.at[dstb.at[0]], ss1).wait()
        return carry

    lax.fori_loop(0, nchunks, chunk_body, 0)
    plsc.subcore_barrier()
    for k in range(4):
        pltpu.sync_copy(acc_sh.at[pl.ds(s * T_SL2 + k * BATCH, BATCH)],
                        out_hbm.at[c, pl.ds(s * T_SL2 + k * BATCH, BATCH)])
    pltpu.sync_copy(acc_sh.at[pl.ds(s * T_SL2 + 4 * BATCH, T_SL2 - 4 * BATCH)],
                    out_hbm.at[c, pl.ds(s * T_SL2 + 4 * BATCH,
                                        T_SL2 - 4 * BATCH)])


_sck2 = functools.partial(
    pl.kernel,
    out_type=jax.ShapeDtypeStruct((NC, T_ACC, D), jnp.float32),
    mesh=_mesh,
    compiler_params=_sc_params,
    scratch_types=[
        pltpu.VMEM((T_PAD,), jnp.float32),
        pltpu.VMEM((CH_ROWS, 128), jnp.int32),
        pltpu.VMEM((CH_ROWS, 128), jnp.int32),
        pltpu.VMEM((CH_ROWS, 128), jnp.float32),
        pltpu.VMEM((BATCH, D), jnp.float32),
        pltpu.VMEM((BATCH, D), jnp.float32),
        pltpu.VMEM_SHARED((T_ACC, D), jnp.float32),
        pltpu.SemaphoreType.DMA,
        pltpu.SemaphoreType.DMA,
        pltpu.SemaphoreType.DMA,
        pltpu.SemaphoreType.DMA,
    ],
)(_sck2_body)


def _bn_body(o2_ref, xt_ref, g_ref, bt_ref, out_ref):
    agg = o2_ref[0, :N_TASTE, :] + o2_ref[1, :N_TASTE, :]
    y = jnp.maximum(agg, 0.0) + xt_ref[...]
    mean = jnp.mean(y, axis=0, keepdims=True)
    var = jnp.mean(jnp.square(y - mean), axis=0, keepdims=True)
    yn = (y - mean) / jnp.sqrt(var + 1e-5) * g_ref[...] + bt_ref[...]
    out_ref[...] = jnp.maximum(yn, 0.0)


def _bn(out2, x_taste, g2, bt2):
    return pl.pallas_call(
        _bn_body,
        out_shape=jax.ShapeDtypeStruct((N_TASTE, D), jnp.float32),
    )(out2, x_taste, g2, bt2)


def kernel(x_ingredient, x_taste, edge_src, edge_dst, W_ing, b_ing,
           W_taste, b_taste, att_src, att_dst, Wk, bk, q, gamma, beta):
    del Wk, bk, q

    pad = jnp.zeros((E_PAD - E,), jnp.int32)
    src2 = jnp.concatenate([edge_src, pad]).reshape(ROWS_CH, 128)
    dst2 = jnp.concatenate([edge_dst, pad]).reshape(ROWS_CH, 128)

    a_src, amax_s = _score(
        x_ingredient, W_ing, b_ing.reshape(1, D), att_src.reshape(1, D),
        N_ING, 4000)
    a_src = a_src.reshape(N_ING)
    a_dst, amax_d = _score(
        x_taste, W_taste, b_taste.reshape(1, D), att_dst.reshape(1, D),
        N_TASTE, 2000)
    a_dst = a_dst.reshape(N_TASTE)
    h_ing = _h_proj(x_ingredient, W_ing, b_ing.reshape(1, D), N_ING, 4000)

    mb = amax_s[0, 0] + amax_d[0, 0]
    mb = jnp.where(mb >= 0.0, mb, 0.2 * mb)
    m_vec = jnp.full((L,), mb, jnp.float32)

    ex2, denom2 = _sck1(src2, dst2, a_src, a_dst, m_vec)
    denom3 = denom2.reshape(NC, T_PAD // 128, 128)
    out2 = _sck2(src2, dst2, ex2, denom2, denom3, h_ing)
    out_t = _bn(out2, x_taste, gamma.reshape(1, D), beta.reshape(1, D))
    return (x_ingredient, out_t)


# speedup vs baseline: 1.0298x; 1.0104x over previous
import functools

import jax
import jax.numpy as jnp
from jax import lax
from jax.experimental import pallas as pl
from jax.experimental.pallas import tpu as pltpu
from jax.experimental.pallas import tpu_sc as plsc

N_ING = 100000
N_TASTE = 10000
E = 625000
D = 128

NC = 2
NS = 16
NW = NC * NS
L = 16

CHUNK = 2048
N_CHUNKS = (E + CHUNK - 1) // CHUNK
E_PAD = N_CHUNKS * CHUNK
ROWS_CH = E_PAD // 128
CH_ROWS = CHUNK // 128
T_PAD = 10240
T_SL = T_PAD // NS
T_ACC = 10112
T_SL2 = T_ACC // NS
BATCH = 128
NB = CHUNK // BATCH

_mesh = plsc.VectorSubcoreMesh(core_axis_name="c", subcore_axis_name="s")
_sc_params = pltpu.CompilerParams(needs_layout_passes=False)


def _score_body(x_ref, w_ref, b_ref, att_ref, a_ref, amax_ref):
    i = pl.program_id(0)
    vr = jax.lax.dot_general(att_ref[...], w_ref[...], (((1,), (1,)), ((), ())),
                             preferred_element_type=jnp.float32,
                             precision=jax.lax.Precision.HIGHEST)
    c0 = jnp.sum(b_ref[...] * att_ref[...])
    a = jnp.sum(x_ref[...] * vr, axis=1)
    a = a + c0
    a_ref[...] = a.reshape(a_ref.shape)
    m = jnp.max(a)

    @pl.when(i == 0)
    def _():
        amax_ref[0, 0] = m

    @pl.when(i > 0)
    def _():
        amax_ref[0, 0] = jnp.maximum(amax_ref[0, 0], m)


def _score(x, w, b2, att2, n_rows, blk):
    grid = (n_rows // blk,)
    return pl.pallas_call(
        _score_body,
        grid=grid,
        in_specs=[
            pl.BlockSpec((blk, D), lambda i: (i, 0)),
            pl.BlockSpec((D, D), lambda i: (0, 0)),
            pl.BlockSpec((1, D), lambda i: (0, 0)),
            pl.BlockSpec((1, D), lambda i: (0, 0)),
        ],
        out_specs=[
            pl.BlockSpec((1, 1, blk), lambda i: (i, 0, 0)),
            pl.BlockSpec((1, 1), lambda i: (0, 0),
                         memory_space=pltpu.SMEM),
        ],
        out_shape=[
            jax.ShapeDtypeStruct((n_rows // blk, 1, blk), jnp.float32),
            jax.ShapeDtypeStruct((1, 1), jnp.float32),
        ],
    )(x, w, b2, att2)


def _h_body(x_ref, w_ref, b_ref, h_ref):
    h_ref[...] = jnp.dot(x_ref[...], w_ref[...],
                         preferred_element_type=jnp.float32,
                         precision=jax.lax.Precision.HIGHEST) + b_ref[...]


def _h_proj(x, w, b2, n_rows, blk):
    return pl.pallas_call(
        _h_body,
        grid=(n_rows // blk,),
        in_specs=[
            pl.BlockSpec((blk, D), lambda i: (i, 0)),
            pl.BlockSpec((D, D), lambda i: (0, 0)),
            pl.BlockSpec((1, D), lambda i: (0, 0)),
        ],
        out_specs=pl.BlockSpec((blk, D), lambda i: (i, 0)),
        out_shape=jax.ShapeDtypeStruct((n_rows, D), jnp.float32),
    )(x, w, b2)


def _sck1_body(src_hbm, dst_hbm, asrc_hbm, adst_hbm, m_hbm,
               ex_hbm, denom_hbm,
               asrc_v, adst_v, m_v, srcb, dstb, exb, zbuf, denom_sh,
               se, sdn):
    c = lax.axis_index("c")
    s = lax.axis_index("s")
    wid = s * NC + c

    pltpu.sync_copy(asrc_hbm, asrc_v)
    pltpu.sync_copy(adst_hbm, adst_v)
    pltpu.sync_copy(m_hbm, m_v)
    m = m_v[...][0]

    z16 = jnp.zeros((L,), jnp.float32)
    for k in range(T_SL // L):
        zbuf[pl.ds(k * L, L)] = z16
    pltpu.sync_copy(zbuf, denom_sh.at[pl.ds(s * T_SL, T_SL)])
    plsc.subcore_barrier()

    nchunks = (N_CHUNKS - wid + NW - 1) // NW

    def chunk_body(ci, carry):
        ch = wid + ci * NW
        base_row = ch * CH_ROWS
        pltpu.sync_copy(src_hbm.at[pl.ds(base_row, CH_ROWS)], srcb)
        pltpu.sync_copy(dst_hbm.at[pl.ds(base_row, CH_ROWS)], dstb)

        def row_body(row, rcarry):
            for col in range(8):
                si = srcb[row, pl.ds(col * L, L)]
                di = dstb[row, pl.ds(col * L, L)]
                av = plsc.load_gather(asrc_v, [si])
                dv = plsc.load_gather(adst_v, [di])
                al = av + dv
                al = jnp.where(al >= 0.0, al, 0.2 * al)
                ex = jnp.exp(al - m)
                gidx = (base_row + row) * 128 + col * L + lax.iota(jnp.int32, L)
                ex = jnp.where(gidx < E, ex, 0.0)
                exb[row, pl.ds(col * L, L)] = ex
            return rcarry

        lax.fori_loop(0, CH_ROWS, row_body, 0)
        exd = pltpu.async_copy(exb, ex_hbm.at[pl.ds(base_row, CH_ROWS)], se)
        dds = [pltpu.async_copy(exb.at[row], denom_sh.at[dstb.at[row]],
                                sdn, add=True) for row in range(CH_ROWS)]
        exd.wait()
        for dsc in dds:
            dsc.wait()
        return carry

    lax.fori_loop(0, nchunks, chunk_body, 0)
    plsc.subcore_barrier()
    pltpu.sync_copy(denom_sh.at[pl.ds(s * T_SL, T_SL)],
                    denom_hbm.at[c, pl.ds(s * T_SL, T_SL)])


_sck1 = functools.partial(
    pl.kernel,
    out_type=[
        jax.ShapeDtypeStruct((ROWS_CH, 128), jnp.float32),
        jax.ShapeDtypeStruct((NC, T_PAD), jnp.float32),
    ],
    mesh=_mesh,
    compiler_params=_sc_params,
    scratch_types=[
        pltpu.VMEM((N_ING,), jnp.float32),
        pltpu.VMEM((N_TASTE,), jnp.float32),
        pltpu.VMEM((L,), jnp.float32),
        pltpu.VMEM((CH_ROWS, 128), jnp.int32),
        pltpu.VMEM((CH_ROWS, 128), jnp.int32),
        pltpu.VMEM((CH_ROWS, 128), jnp.float32),
        pltpu.VMEM((T_SL,), jnp.float32),
        pltpu.VMEM_SHARED((T_PAD,), jnp.float32),
        pltpu.SemaphoreType.DMA,
        pltpu.SemaphoreType.DMA,
    ],
)(_sck1_body)


def _scale_batch(rows_ref, exb, b):
    @plsc.parallel_loop(0, BATCH // L, unroll=2)
    def _(g):
        sv = exb[b, pl.ds(g * L, L)]
        for k in range(L):
            svb = jnp.full((L,), sv[k], jnp.float32)
            e = g * L + k
            for j in range(8):
                rows_ref[e, pl.ds(j * L, L)] = rows_ref[e, pl.ds(j * L, L)] * svb


def _sck2_body(src_hbm, dst_hbm, ex_hbm, denom_hbm, denom3_hbm, h_hbm,
               out_hbm,
               r_v, srcb, dstb, exb, rows0, rows1, acc_sh,
               sg0, sg1, ss0, ss1):
    c = lax.axis_index("c")
    s = lax.axis_index("s")
    wid = s * NC + c

    pltpu.sync_copy(denom_hbm.at[0], r_v)
    pltpu.sync_copy(denom3_hbm.at[1], rows1.at[pl.ds(0, T_PAD // 128)])

    @plsc.parallel_loop(0, T_PAD // 128, unroll=2)
    def _(row):
        for col in range(8):
            off = row * 128 + col * L
            dsum = r_v[pl.ds(off, L)] + rows1[row, pl.ds(col * L, L)]
            r_v[pl.ds(off, L)] = 1.0 / jnp.maximum(dsum, 1e-16)

    z16 = jnp.zeros((L,), jnp.float32)

    def zrow_body(e, carry):
        for j in range(8):
            rows0[e, pl.ds(j * L, L)] = z16
        return carry

    lax.fori_loop(0, BATCH, zrow_body, 0)
    for k in range(4):
        pltpu.sync_copy(rows0, acc_sh.at[pl.ds(s * T_SL2 + k * BATCH, BATCH)])
    pltpu.sync_copy(rows0.at[pl.ds(0, T_SL2 - 4 * BATCH)],
                    acc_sh.at[pl.ds(s * T_SL2 + 4 * BATCH, T_SL2 - 4 * BATCH)])
    plsc.subcore_barrier()

    nchunks = (N_CHUNKS - wid + NW - 1) // NW

    def chunk_body(ci, carry):
        ch = wid + ci * NW
        base_row = ch * CH_ROWS
        pltpu.sync_copy(src_hbm.at[pl.ds(base_row, CH_ROWS)], srcb)
        pltpu.sync_copy(dst_hbm.at[pl.ds(base_row, CH_ROWS)], dstb)
        pltpu.sync_copy(ex_hbm.at[pl.ds(base_row, CH_ROWS)], exb)
        def srow_body(row, rcarry):
            for col in range(8):
                di = dstb[row, pl.ds(col * L, L)]
                rv = plsc.load_gather(r_v, [di])
                exb[row, pl.ds(col * L, L)] = exb[row, pl.ds(col * L, L)] * rv
            return rcarry

        lax.fori_loop(0, CH_ROWS, srow_body, 0)
        pltpu.async_copy(h_hbm.at[srcb.at[0]], rows0, sg0)

        def pair_body(i, pcarry):
            b0 = i * 2
            pltpu.make_async_copy(h_hbm.at[srcb.at[0]], rows0, sg0).wait()

            @pl.when(i > 0)
            def _():
                pltpu.make_async_copy(rows1, acc_sh.at[dstb.at[0]], ss1).wait()

            pltpu.async_copy(h_hbm.at[srcb.at[b0 + 1]], rows1, sg1)
            _scale_batch(rows0, exb, b0)
            pltpu.async_copy(rows0, acc_sh.at[dstb.at[b0]], ss0, add=True)
            pltpu.make_async_copy(h_hbm.at[srcb.at[0]], rows1, sg1).wait()

            @pl.when(i + 1 < NB // 2)
            def _():
                pltpu.make_async_copy(rows0, acc_sh.at[dstb.at[0]], ss0).wait()
                pltpu.async_copy(h_hbm.at[srcb.at[b0 + 2]], rows0, sg0)

            _scale_batch(rows1, exb, b0 + 1)
            pltpu.async_copy(rows1, acc_sh.at[dstb.at[b0 + 1]], ss1, add=True)
            return pcarry

        lax.fori_loop(0, NB // 2, pair_body, 0)
        pltpu.make_async_copy(rows0, acc_sh.at[dstb.at[0]], ss0).wait()
        pltpu.make_async_copy(rows1, acc_sh.at[dstb.at[0]], ss1).wait()
        return carry

    lax.fori_loop(0, nchunks, chunk_body, 0)
    plsc.subcore_barrier()
    for k in range(4):
        pltpu.sync_copy(acc_sh.at[pl.ds(s * T_SL2 + k * BATCH, BATCH)],
                        out_hbm.at[c, pl.ds(s * T_SL2 + k * BATCH, BATCH)])
    pltpu.sync_copy(acc_sh.at[pl.ds(s * T_SL2 + 4 * BATCH, T_SL2 - 4 * BATCH)],
                    out_hbm.at[c, pl.ds(s * T_SL2 + 4 * BATCH,
                                        T_SL2 - 4 * BATCH)])


_sck2 = functools.partial(
    pl.kernel,
    out_type=jax.ShapeDtypeStruct((NC, T_ACC, D), jnp.float32),
    mesh=_mesh,
    compiler_params=_sc_params,
    scratch_types=[
        pltpu.VMEM((T_PAD,), jnp.float32),
        pltpu.VMEM((CH_ROWS, 128), jnp.int32),
        pltpu.VMEM((CH_ROWS, 128), jnp.int32),
        pltpu.VMEM((CH_ROWS, 128), jnp.float32),
        pltpu.VMEM((BATCH, D), jnp.float32),
        pltpu.VMEM((BATCH, D), jnp.float32),
        pltpu.VMEM_SHARED((T_ACC, D), jnp.float32),
        pltpu.SemaphoreType.DMA,
        pltpu.SemaphoreType.DMA,
        pltpu.SemaphoreType.DMA,
        pltpu.SemaphoreType.DMA,
    ],
)(_sck2_body)


def _bn_body(o2_ref, xt_ref, g_ref, bt_ref, out_ref):
    agg = o2_ref[0, :N_TASTE, :] + o2_ref[1, :N_TASTE, :]
    y = jnp.maximum(agg, 0.0) + xt_ref[...]
    mean = jnp.mean(y, axis=0, keepdims=True)
    var = jnp.mean(jnp.square(y - mean), axis=0, keepdims=True)
    yn = (y - mean) / jnp.sqrt(var + 1e-5) * g_ref[...] + bt_ref[...]
    out_ref[...] = jnp.maximum(yn, 0.0)


def _bn(out2, x_taste, g2, bt2):
    return pl.pallas_call(
        _bn_body,
        out_shape=jax.ShapeDtypeStruct((N_TASTE, D), jnp.float32),
    )(out2, x_taste, g2, bt2)


def kernel(x_ingredient, x_taste, edge_src, edge_dst, W_ing, b_ing,
           W_taste, b_taste, att_src, att_dst, Wk, bk, q, gamma, beta):
    del Wk, bk, q

    pad = jnp.zeros((E_PAD - E,), jnp.int32)
    src2 = jnp.concatenate([edge_src, pad]).reshape(ROWS_CH, 128)
    dst2 = jnp.concatenate([edge_dst, pad]).reshape(ROWS_CH, 128)

    a_src, amax_s = _score(
        x_ingredient, W_ing, b_ing.reshape(1, D), att_src.reshape(1, D),
        N_ING, 4000)
    a_src = a_src.reshape(N_ING)
    a_dst, amax_d = _score(
        x_taste, W_taste, b_taste.reshape(1, D), att_dst.reshape(1, D),
        N_TASTE, 2000)
    a_dst = a_dst.reshape(N_TASTE)
    h_ing = _h_proj(x_ingredient, W_ing, b_ing.reshape(1, D), N_ING, 4000)

    mb = amax_s[0, 0] + amax_d[0, 0]
    mb = jnp.where(mb >= 0.0, mb, 0.2 * mb)
    m_vec = jnp.full((L,), mb, jnp.float32)

    ex2, denom2 = _sck1(src2, dst2, a_src, a_dst, m_vec)
    denom3 = denom2.reshape(NC, T_PAD // 128, 128)
    out2 = _sck2(src2, dst2, ex2, denom2, denom3, h_ing)
    out_t = _bn(out2, x_taste, gamma.reshape(1, D), beta.reshape(1, D))
    return (x_ingredient, out_t)
